# baseline (speedup 1.0000x reference)
import jax
import jax.numpy as jnp
from jax.experimental import pallas as pl

_B = 16
_HW = {8: 64 * 64, 16: 32 * 32, 32: 16 * 16}
_THR = 0.5


def _body(*refs):
    (t0c8, t1c8, t0b8, t0t8, t1b8, t1t8,
     t0c16, t1c16, t0b16, t0t16, t1b16, t1t16,
     t0c32, t0b32, t0t32) = refs[:15]
    outs = refs[15:]

    for n in range(_B):
        def mask_of(c0, c1):
            mx = jnp.max(c0[n], axis=0)
            if c1 is not None:
                mx = jnp.maximum(mx, jnp.max(c1[n], axis=0))
            return (mx > _THR).astype(jnp.float32)[None, :]

        m8 = mask_of(t0c8, t1c8)
        m16 = mask_of(t0c16, t1c16)
        m32 = mask_of(t0c32, None)
        for k, src in enumerate((t0b8, t0t8, t1b8, t1t8)):
            outs[4 * n + k][...] = src[n] * m8
        for k, src in enumerate((t0b16, t0t16, t1b16, t1t16)):
            outs[64 + 4 * n + k][...] = src[n] * m16
        for k, src in enumerate((t0b32, t0t32)):
            outs[128 + 2 * n + k][...] = src[n] * m32


def kernel(t0_cls_s8, t0_cls_s16, t0_cls_s32,
           t0_box_s8, t0_box_s16, t0_box_s32,
           t0_ctr_s8, t0_ctr_s16, t0_ctr_s32,
           t1_cls_s8, t1_cls_s16,
           t1_box_s8, t1_box_s16,
           t1_ctr_s8, t1_ctr_s16):
    def flat(x):
        n, c, h, w = x.shape
        return x.reshape(n, c, h * w)

    ins = [flat(t0_cls_s8), flat(t1_cls_s8),
           flat(t0_box_s8), flat(t0_ctr_s8), flat(t1_box_s8), flat(t1_ctr_s8),
           flat(t0_cls_s16), flat(t1_cls_s16),
           flat(t0_box_s16), flat(t0_ctr_s16), flat(t1_box_s16), flat(t1_ctr_s16),
           flat(t0_cls_s32), flat(t0_box_s32), flat(t0_ctr_s32)]

    out_shapes = []
    for s, chans in ((8, (4, 1, 4, 1)), (16, (4, 1, 4, 1)), (32, (4, 1))):
        for _ in range(_B):
            for c in chans:
                out_shapes.append(jax.ShapeDtypeStruct((c, _HW[s]), jnp.float32))

    outs = pl.pallas_call(_body, out_shape=out_shapes)(*ins)

    dims = {8: (64, 64), 16: (32, 32), 32: (16, 16)}
    result = []
    i = 0
    for s, chans in ((8, (4, 1, 4, 1)), (16, (4, 1, 4, 1)), (32, (4, 1))):
        h, w = dims[s]
        for _ in range(_B):
            for c in chans:
                result.append(outs[i].reshape(c, h, w))
                i += 1
    return tuple(result)
